# baseline (speedup 1.0000x reference)
import functools

import jax
import jax.numpy as jnp
from jax import lax
from jax.experimental import pallas as pl
from jax.experimental.pallas import tpu as pltpu
from jax.experimental.pallas import tpu_sc as plsc

VOCAB = 1000
EMB = 1000
BATCH = 1024
SEQ = 50
SPLITS = 5
SEQ_K = SEQ // SPLITS

NUM_CORES = 2
NUM_SUBCORES = 16
NW = NUM_CORES * NUM_SUBCORES
B_PER_W = BATCH // NW


def _gather_body(table_hbm, idx_hbm, out_hbm, idx_v, rows0, rows1, sem0, sem1):
    wid = lax.axis_index("s") * NUM_CORES + lax.axis_index("c")
    base = wid * B_PER_W
    pltpu.sync_copy(idx_hbm.at[wid], idx_v)

    bufs = ((rows0, sem0), (rows1, sem1))

    def start(s, p):
        rows, sem = bufs[p]
        pltpu.async_copy(table_hbm.at[idx_v.at[s]], rows, sem)

    def drain(s, p):
        rows, sem = bufs[p]
        pltpu.make_async_copy(table_hbm.at[idx_v.at[s]], rows, sem).wait()
        pltpu.sync_copy(rows, out_hbm.at[s, pl.ds(base, B_PER_W)])

    start(0, 0)

    @pl.loop(0, SEQ_K, step=2)
    def _(s):
        start(s + 1, 1)
        drain(s, 0)

        @pl.when(s + 2 < SEQ_K)
        def _():
            start(s + 2, 0)

        drain(s + 1, 1)


_mesh = plsc.VectorSubcoreMesh(core_axis_name="c", subcore_axis_name="s")

EMBP = 1024

_gather_call = pl.kernel(
    _gather_body,
    out_type=jax.ShapeDtypeStruct((SEQ_K, BATCH, EMBP), jnp.float32),
    mesh=_mesh,
    scratch_types=[
        pltpu.VMEM((SEQ_K, B_PER_W), jnp.int32),
        pltpu.VMEM((B_PER_W, EMBP), jnp.float32),
        pltpu.VMEM((B_PER_W, EMBP), jnp.float32),
        pltpu.SemaphoreType.DMA,
        pltpu.SemaphoreType.DMA,
    ],
)


@jax.jit
def kernel(token_ids, token_embedding):
    idxT = (
        token_ids.astype(jnp.int32)
        .T.reshape(SEQ, NW, B_PER_W)
        .transpose(1, 0, 2)
    )
    table = jnp.pad(token_embedding, ((0, 0), (0, EMBP - EMB)))
    pieces = []
    for i in range(SPLITS):
        out4 = _gather_call(table, idxT[:, i * SEQ_K:(i + 1) * SEQ_K, :])
        pieces.append(out4[:, :, :EMB].transpose(1, 0, 2))
    return jnp.concatenate(pieces, axis=1)


# speedup vs baseline: 1.0071x; 1.0071x over previous
import functools

import jax
import jax.numpy as jnp
from jax import lax
from jax.experimental import pallas as pl
from jax.experimental.pallas import tpu as pltpu
from jax.experimental.pallas import tpu_sc as plsc

VOCAB = 1000
EMB = 1000
BATCH = 1024
SEQ = 50

NUM_CORES = 2
NUM_SUBCORES = 16
NW = NUM_CORES * NUM_SUBCORES
B_PER_W = BATCH // NW


def _gather_body(table_hbm, idx_hbm, out_hbm, idx_v,
                 rows0, rows1, rows2, gs0, gs1, gs2, ws0, ws1, ws2):
    wid = lax.axis_index("s") * NUM_CORES + lax.axis_index("c")
    base = wid * B_PER_W
    pltpu.sync_copy(idx_hbm.at[wid], idx_v)

    bufs = ((rows0, gs0, ws0), (rows1, gs1, ws1), (rows2, gs2, ws2))

    def start(s, p):
        rows, gsem, _ = bufs[p]
        pltpu.async_copy(table_hbm.at[idx_v.at[s]], rows, gsem)

    def wait_gather(s, p):
        rows, gsem, _ = bufs[p]
        pltpu.make_async_copy(table_hbm.at[idx_v.at[s]], rows, gsem).wait()

    def start_write(s, p):
        rows, _, wsem = bufs[p]
        pltpu.async_copy(rows, out_hbm.at[s, pl.ds(base, B_PER_W)], wsem)

    def wait_write(s, p):
        rows, _, wsem = bufs[p]
        pltpu.make_async_copy(
            rows, out_hbm.at[s, pl.ds(base, B_PER_W)], wsem
        ).wait()

    start(0, 0)
    start(1, 1)

    @pl.loop(0, SEQ - 2, step=3)
    def _(s):
        for k in range(3):
            sk = s + k

            @pl.when(sk >= 3)
            def _():
                wait_write(sk - 3, k)

            start(sk + 2, k)
            wait_gather(sk, k)
            start_write(sk, k)

    wait_write(SEQ - 5, 0)
    wait_gather(SEQ - 2, 0)
    start_write(SEQ - 2, 0)
    wait_write(SEQ - 4, 1)
    wait_gather(SEQ - 1, 1)
    start_write(SEQ - 1, 1)
    wait_write(SEQ - 3, 2)
    wait_write(SEQ - 2, 0)
    wait_write(SEQ - 1, 1)


_mesh = plsc.VectorSubcoreMesh(core_axis_name="c", subcore_axis_name="s")

EMBP = 1024

_gather_call = pl.kernel(
    _gather_body,
    out_type=jax.ShapeDtypeStruct((SEQ, BATCH, EMBP), jnp.float32),
    mesh=_mesh,
    scratch_types=[
        pltpu.VMEM((SEQ, B_PER_W), jnp.int32),
        pltpu.VMEM((B_PER_W, EMBP), jnp.float32),
        pltpu.VMEM((B_PER_W, EMBP), jnp.float32),
        pltpu.VMEM((B_PER_W, EMBP), jnp.float32),
        pltpu.SemaphoreType.DMA,
        pltpu.SemaphoreType.DMA,
        pltpu.SemaphoreType.DMA,
        pltpu.SemaphoreType.DMA,
        pltpu.SemaphoreType.DMA,
        pltpu.SemaphoreType.DMA,
    ],
)


@jax.jit
def kernel(token_ids, token_embedding):
    idxT = (
        token_ids.astype(jnp.int32)
        .T.reshape(SEQ, NW, B_PER_W)
        .transpose(1, 0, 2)
    )
    table = jnp.pad(token_embedding, ((0, 0), (0, EMBP - EMB)))
    out4 = _gather_call(table, idxT)
    return out4[:, :, :EMB].transpose(1, 0, 2)


# speedup vs baseline: 1.0124x; 1.0052x over previous
import functools

import jax
import jax.numpy as jnp
from jax import lax
from jax.experimental import pallas as pl
from jax.experimental.pallas import tpu as pltpu
from jax.experimental.pallas import tpu_sc as plsc

VOCAB = 1000
EMB = 1000
BATCH = 1024
SEQ = 50

NUM_CORES = 2
NUM_SUBCORES = 16
NW = NUM_CORES * NUM_SUBCORES
B_PER_W = BATCH // NW


def _gather_body(table_hbm, idx_hbm, out_hbm, idx_v, rows0, rows1, sem0, sem1):
    wid = lax.axis_index("s") * NUM_CORES + lax.axis_index("c")
    base = wid * B_PER_W
    pltpu.sync_copy(idx_hbm.at[wid], idx_v)

    bufs = ((rows0, sem0), (rows1, sem1))

    def start(s, p):
        rows, sem = bufs[p]
        pltpu.async_copy(table_hbm.at[idx_v.at[s]], rows, sem)

    def drain(s, p):
        rows, sem = bufs[p]
        pltpu.make_async_copy(table_hbm.at[idx_v.at[s]], rows, sem).wait()
        pltpu.sync_copy(rows, out_hbm.at[s, pl.ds(base, B_PER_W)])

    start(0, 0)

    @pl.loop(0, SEQ, step=2)
    def _(s):
        start(s + 1, 1)
        drain(s, 0)

        @pl.when(s + 2 < SEQ)
        def _():
            start(s + 2, 0)

        drain(s + 1, 1)


_mesh = plsc.VectorSubcoreMesh(core_axis_name="c", subcore_axis_name="s")

EMBP = 1024

_gather_call = pl.kernel(
    _gather_body,
    out_type=jax.ShapeDtypeStruct((SEQ, BATCH, EMBP), jnp.float32),
    mesh=_mesh,
    scratch_types=[
        pltpu.VMEM((SEQ, B_PER_W), jnp.int32),
        pltpu.VMEM((B_PER_W, EMBP), jnp.float32),
        pltpu.VMEM((B_PER_W, EMBP), jnp.float32),
        pltpu.SemaphoreType.DMA,
        pltpu.SemaphoreType.DMA,
    ],
)


@jax.jit
def kernel(token_ids, token_embedding):
    idxT = (
        token_ids.astype(jnp.int32)
        .T.reshape(SEQ, NW, B_PER_W)
        .transpose(1, 0, 2)
    )
    table = jnp.pad(token_embedding, ((0, 0), (0, EMBP - EMB)))
    out4 = _gather_call(table, idxT)
    return out4[:, :, :EMB].transpose(1, 0, 2)


# speedup vs baseline: 1.0129x; 1.0006x over previous
import jax
import jax.numpy as jnp
from jax import lax
from jax.experimental import pallas as pl
from jax.experimental.pallas import tpu as pltpu
from jax.experimental.pallas import tpu_sc as plsc

VOCAB = 1000
EMB = 1000
BATCH = 1024
SEQ = 50

NUM_CORES = 2
NUM_SUBCORES = 16
NW = NUM_CORES * NUM_SUBCORES
B_PER_W = BATCH // NW


def _gather_body(table_hbm, idx_hbm, out_hbm, idx_v, rows0, rows1, sem0, sem1):
    wid = lax.axis_index("s") * NUM_CORES + lax.axis_index("c")
    base = wid * B_PER_W
    pltpu.sync_copy(idx_hbm.at[wid], idx_v)

    bufs = ((rows0, sem0), (rows1, sem1))

    def start(s, p):
        rows, sem = bufs[p]
        pltpu.async_copy(table_hbm.at[idx_v.at[s]], rows, sem)

    def drain(s, p):
        rows, sem = bufs[p]
        pltpu.make_async_copy(table_hbm.at[idx_v.at[s]], rows, sem).wait()
        pltpu.sync_copy(rows, out_hbm.at[s, pl.ds(base, B_PER_W)])

    start(0, 0)

    @pl.loop(0, SEQ, step=2)
    def _(s):
        start(s + 1, 1)
        drain(s, 0)

        @pl.when(s + 2 < SEQ)
        def _():
            start(s + 2, 0)

        drain(s + 1, 1)


_mesh = plsc.VectorSubcoreMesh(core_axis_name="c", subcore_axis_name="s")

EMBP = 1024

_gather_call = pl.kernel(
    _gather_body,
    out_type=jax.ShapeDtypeStruct((SEQ, BATCH, EMBP), jnp.float32),
    mesh=_mesh,
    scratch_types=[
        pltpu.VMEM((SEQ, B_PER_W), jnp.int32),
        pltpu.VMEM((B_PER_W, EMBP), jnp.float32),
        pltpu.VMEM((B_PER_W, EMBP), jnp.float32),
        pltpu.SemaphoreType.DMA,
        pltpu.SemaphoreType.DMA,
    ],
)


@jax.jit
def kernel(token_ids, token_embedding):
    idxT = (
        token_ids.astype(jnp.int32)
        .T.reshape(SEQ, NW, B_PER_W)
        .transpose(1, 0, 2)
    )
    table = jnp.pad(token_embedding, ((0, 0), (0, EMBP - EMB)))
    out4 = _gather_call(table, idxT)
    return out4[:, :, :EMB].transpose(1, 0, 2)
